# baseline (speedup 1.0000x reference)
import functools

import jax
import jax.numpy as jnp
from jax import lax
from jax.experimental import pallas as pl
from jax.experimental.pallas import tpu as pltpu
from jax.experimental.pallas import tpu_sc as plsc

NUM_EMB = 1024
DIM = 64
ROWS = 16 * 576
BLK = 1024


def _argmin_body(x_ref, embT_ref, idx_ref):
    x = x_ref[...]
    embT = embT_ref[...]
    se = jnp.sum(embT * embT, axis=0, keepdims=True)
    sx = jnp.sum(x * x, axis=1, keepdims=True)
    dot = jnp.dot(x, embT, preferred_element_type=jnp.float32)
    d = (sx + se) - 2.0 * dot
    m = jnp.min(d, axis=1, keepdims=True)
    iota = lax.broadcasted_iota(jnp.int32, d.shape, 1)
    cand = jnp.where(d == m, iota, NUM_EMB)
    idx_ref[...] = jnp.min(cand, axis=1)


def _compute_indices(flat, embT):
    return pl.pallas_call(
        _argmin_body,
        grid=(ROWS // BLK,),
        in_specs=[
            pl.BlockSpec((BLK, DIM), lambda i: (i, 0)),
            pl.BlockSpec((DIM, NUM_EMB), lambda i: (0, 0)),
        ],
        out_specs=pl.BlockSpec((BLK,), lambda i: (i,)),
        out_shape=jax.ShapeDtypeStruct((ROWS,), jnp.int32),
    )(flat, embT)


GDIM = 128


@functools.cache
def _make_gather():
    info = plsc.get_sparse_core_info()
    nw = info.num_cores * info.num_subcores
    b_per_w = ROWS // nw

    @functools.partial(
        pl.kernel,
        out_type=jax.ShapeDtypeStruct((ROWS, GDIM), jnp.float32),
        mesh=plsc.VectorSubcoreMesh(core_axis_name="c", subcore_axis_name="s"),
        scratch_types=[
            pltpu.VMEM((b_per_w,), jnp.int32),
            pltpu.VMEM((b_per_w, GDIM), jnp.float32),
            pltpu.SemaphoreType.DMA,
        ],
    )
    def _gather_rows(emb_hbm, idx_hbm, out_hbm, idx_v, rows_v, sem):
        wid = lax.axis_index("s") * info.num_cores + lax.axis_index("c")
        base = wid * b_per_w
        pltpu.sync_copy(idx_hbm.at[pl.ds(base, b_per_w)], idx_v)
        pltpu.async_copy(emb_hbm.at[idx_v], rows_v, sem).wait()
        pltpu.sync_copy(rows_v, out_hbm.at[pl.ds(base, b_per_w)])

    return _gather_rows


def kernel(inputs, emb_weight):
    b, s, c = inputs.shape
    flat = inputs.reshape(b * s, c)
    idx = _compute_indices(flat, emb_weight.T)
    emb_pad = jnp.pad(emb_weight, ((0, 0), (0, GDIM - DIM)))
    quantized = _make_gather()(emb_pad, idx)[:, :DIM]
    return (quantized.reshape(b, s, c), idx.reshape(b, s))


# speedup vs baseline: 1.0174x; 1.0174x over previous
import functools

import jax
import jax.numpy as jnp
from jax import lax
from jax.experimental import pallas as pl
from jax.experimental.pallas import tpu as pltpu
from jax.experimental.pallas import tpu_sc as plsc

NUM_EMB = 1024
DIM = 64
ROWS = 16 * 576
BLK = 1024


def _argmin_body(x_ref, emb_ref, idx_ref, se_ref, iota_ref):
    @pl.when(pl.program_id(0) == 0)
    def _():
        e = emb_ref[...]
        se_ref[...] = jnp.sum(e * e, axis=1)[None, :]
        iota_ref[...] = lax.broadcasted_iota(
            jnp.int32, (1, NUM_EMB), 1).astype(jnp.float32)

    x = x_ref[...]
    se = se_ref[...]
    sx = jnp.sum(x * x, axis=1, keepdims=True)
    dot = lax.dot_general(x, emb_ref[...], (((1,), (1,)), ((), ())),
                          preferred_element_type=jnp.float32)
    d = (sx + se) - 2.0 * dot
    m = jnp.min(d, axis=1, keepdims=True)
    cand = jnp.where(d == m, iota_ref[...], float(NUM_EMB))
    idx_ref[...] = jnp.min(cand, axis=1).astype(jnp.int32)


def _compute_indices(flat, emb):
    return pl.pallas_call(
        _argmin_body,
        grid=(ROWS // BLK,),
        in_specs=[
            pl.BlockSpec((BLK, DIM), lambda i: (i, 0)),
            pl.BlockSpec((NUM_EMB, DIM), lambda i: (0, 0)),
        ],
        out_specs=pl.BlockSpec((BLK,), lambda i: (i,)),
        out_shape=jax.ShapeDtypeStruct((ROWS,), jnp.int32),
        scratch_shapes=[pltpu.VMEM((1, NUM_EMB), jnp.float32),
                        pltpu.VMEM((1, NUM_EMB), jnp.float32)],
    )(flat, emb)


GDIM = 128


@functools.cache
def _make_gather():
    info = plsc.get_sparse_core_info()
    nw = info.num_cores * info.num_subcores
    b_per_w = ROWS // nw

    @functools.partial(
        pl.kernel,
        out_type=jax.ShapeDtypeStruct((ROWS, GDIM), jnp.float32),
        mesh=plsc.VectorSubcoreMesh(core_axis_name="c", subcore_axis_name="s"),
        scratch_types=[
            pltpu.VMEM((b_per_w,), jnp.int32),
            pltpu.VMEM((b_per_w, GDIM), jnp.float32),
            pltpu.SemaphoreType.DMA,
        ],
    )
    def _gather_rows(emb_hbm, idx_hbm, out_hbm, idx_v, rows_v, sem):
        wid = lax.axis_index("s") * info.num_cores + lax.axis_index("c")
        base = wid * b_per_w
        pltpu.sync_copy(idx_hbm.at[pl.ds(base, b_per_w)], idx_v)
        pltpu.async_copy(emb_hbm.at[idx_v], rows_v, sem).wait()
        pltpu.sync_copy(rows_v, out_hbm.at[pl.ds(base, b_per_w)])

    return _gather_rows


def kernel(inputs, emb_weight):
    b, s, c = inputs.shape
    flat = inputs.reshape(b * s, c)
    idx = _compute_indices(flat, emb_weight)
    emb_pad = jnp.pad(emb_weight, ((0, 0), (0, GDIM - DIM)))
    quantized = _make_gather()(emb_pad, idx)[:, :DIM]
    return (quantized.reshape(b, s, c), idx.reshape(b, s))


# speedup vs baseline: 1.0799x; 1.0614x over previous
import functools

import jax
import jax.numpy as jnp
from jax import lax
from jax.experimental import pallas as pl
from jax.experimental.pallas import tpu as pltpu
from jax.experimental.pallas import tpu_sc as plsc

NUM_EMB = 1024
DIM = 64
ROWS = 16 * 576
BLK = 1024


def _argmin_body(x_ref, emb_ref, idx_ref, se_ref, iota_ref):
    @pl.when(pl.program_id(0) == 0)
    def _():
        e = emb_ref[...]
        se_ref[...] = jnp.sum(e * e, axis=1)[None, :]
        iota_ref[...] = lax.broadcasted_iota(
            jnp.int32, (1, NUM_EMB), 1).astype(jnp.float32)

    x = x_ref[...]
    se = se_ref[...]
    sx = jnp.sum(x * x, axis=1, keepdims=True)
    dot = lax.dot_general(x, emb_ref[...], (((1,), (1,)), ((), ())),
                          preferred_element_type=jnp.float32)
    d = (sx + se) - 2.0 * dot
    m = jnp.min(d, axis=1, keepdims=True)
    cand = jnp.where(d == m, iota_ref[...], float(NUM_EMB))
    idx_ref[...] = jnp.min(cand, axis=1).astype(jnp.int32)


def _compute_indices(flat, emb):
    return pl.pallas_call(
        _argmin_body,
        grid=(ROWS // BLK,),
        in_specs=[
            pl.BlockSpec((BLK, DIM), lambda i: (i, 0)),
            pl.BlockSpec((NUM_EMB, DIM), lambda i: (0, 0)),
        ],
        out_specs=pl.BlockSpec((BLK,), lambda i: (i,)),
        out_shape=jax.ShapeDtypeStruct((ROWS,), jnp.int32),
        scratch_shapes=[pltpu.VMEM((1, NUM_EMB), jnp.float32),
                        pltpu.VMEM((1, NUM_EMB), jnp.float32)],
    )(flat, emb)


GDIM = 128


@functools.cache
def _make_gather():
    info = plsc.get_sparse_core_info()
    nw = info.num_cores * info.num_subcores
    b_per_w = ROWS // nw

    @functools.partial(
        pl.kernel,
        out_type=jax.ShapeDtypeStruct((ROWS, DIM), jnp.float32),
        mesh=plsc.VectorSubcoreMesh(core_axis_name="c", subcore_axis_name="s"),
        scratch_types=[
            pltpu.VMEM((b_per_w,), jnp.int32),
            pltpu.VMEM((b_per_w, DIM), jnp.float32),
            pltpu.VMEM_SHARED((NUM_EMB, DIM), jnp.float32),
            pltpu.SemaphoreType.DMA,
        ],
    )
    def _gather_rows(emb_hbm, idx_hbm, out_hbm, idx_v, rows_v, table_sh, sem):
        sid = lax.axis_index("s")
        wid = sid * info.num_cores + lax.axis_index("c")
        base = wid * b_per_w

        @pl.when(sid == 0)
        def _():
            pltpu.sync_copy(emb_hbm, table_sh)

        pltpu.sync_copy(idx_hbm.at[pl.ds(base, b_per_w)], idx_v)
        plsc.subcore_barrier()
        pltpu.async_copy(table_sh.at[idx_v], rows_v, sem).wait()
        pltpu.sync_copy(rows_v, out_hbm.at[pl.ds(base, b_per_w)])

    return _gather_rows


def kernel(inputs, emb_weight):
    b, s, c = inputs.shape
    flat = inputs.reshape(b * s, c)
    idx = _compute_indices(flat, emb_weight)
    quantized = _make_gather()(emb_weight, idx)
    return (quantized.reshape(b, s, c), idx.reshape(b, s))
